# baseline (speedup 1.0000x reference)
import functools

import jax
import jax.numpy as jnp
from jax.experimental import pallas as pl
from jax.experimental.pallas import tpu as pltpu

_N = 8192
_D_IN = 2048
_D_OUT = 2048
_H1 = 100
_H2 = 50
_H1A = 128
_H2A = 64
_EPS = 1e-5
_CHUNK = 512
_DEPTH = 6


def _rm_dot(a, b):
    return jax.lax.dot_general(
        a, b, (((1,), (1,)), ((), ())),
        preferred_element_type=jnp.float32)


def _mlp_pipeline(x_hbm, w1_ref, b1_ref, w2_ref, b2_ref, w3_ref, b3_ref,
                  m0_ref, v0_ref, m1_ref, v1_ref, m2_ref, v2_ref, out_hbm,
                  xbuf, obuf, insems, outsems, w1a, b1a, w2a, w3a):
    nch = _N // _CHUNK
    f32 = jnp.float32

    s0 = jax.lax.rsqrt(v0_ref[...] + _EPS)
    s1 = jax.lax.rsqrt(v1_ref[...] + _EPS)
    s2 = jax.lax.rsqrt(v2_ref[...] + _EPS)

    w1a[...] = jnp.concatenate(
        [w1_ref[...] * s0, jnp.zeros((_H1A - _H1, _D_IN), f32)], axis=0)
    b1e = b1_ref[...] - _rm_dot(m0_ref[...] * s0, w1_ref[...])
    b1a[...] = jnp.concatenate(
        [b1e, jnp.ones((1, 1), f32), jnp.zeros((1, _H1A - _H1 - 1), f32)],
        axis=1)

    b2e = b2_ref[...] - _rm_dot(w2_ref[...], m1_ref[...] * s1)
    w2_rows = jnp.concatenate(
        [w2_ref[...] * s1, b2e,
         jnp.zeros((_H2, _H1A - _H1 - 1), f32)], axis=1)
    unit_row = (jax.lax.broadcasted_iota(jnp.int32, (1, _H1A), 1) == _H1
                ).astype(f32)
    w2a[...] = jnp.concatenate(
        [w2_rows, unit_row, jnp.zeros((_H2A - _H2 - 1, _H1A), f32)], axis=0)

    b3e = b3_ref[...] - _rm_dot(w3_ref[...], m2_ref[...] * s2)
    w3a[...] = jnp.concatenate(
        [w3_ref[...] * s2, b3e,
         jnp.zeros((_D_OUT, _H2A - _H2 - 1), f32)], axis=1)

    def in_copy(c, slot):
        return pltpu.make_async_copy(
            x_hbm.at[pl.ds(c * _CHUNK, _CHUNK), :], xbuf.at[slot],
            insems.at[slot])

    def out_copy(c, slot):
        return pltpu.make_async_copy(
            obuf.at[slot], out_hbm.at[pl.ds(c * _CHUNK, _CHUNK), :],
            outsems.at[slot])

    for s in range(_DEPTH):
        in_copy(s, s).start()

    for c in range(nch):
        slot = c % _DEPTH
        in_copy(c, slot).wait()
        if c >= _DEPTH:
            out_copy(c - _DEPTH, slot).wait()
        h = jnp.maximum(_rm_dot(xbuf[slot], w1a[...]) + b1a[...], 0.0)
        g = jnp.maximum(_rm_dot(h, w2a[...]), 0.0)
        obuf[slot] = _rm_dot(g, w3a[...])
        out_copy(c, slot).start()
        if c + _DEPTH < nch:
            in_copy(c + _DEPTH, slot).start()

    for c in range(nch - _DEPTH, nch):
        out_copy(c, c % _DEPTH).wait()


@functools.partial(jax.jit, static_argnames=("interpret",))
def kernel(x, W1, b1, W2, b2, W3, b3, bn0_mean, bn0_var, bn1_mean, bn1_var,
           bn2_mean, bn2_var, interpret=False):
    n = x.shape[0]
    any_spec = pl.BlockSpec(memory_space=pl.MemorySpace.ANY)
    vmem = pl.BlockSpec(memory_space=pltpu.MemorySpace.VMEM)

    return pl.pallas_call(
        _mlp_pipeline,
        in_specs=[any_spec] + [vmem] * 12,
        out_specs=any_spec,
        out_shape=jax.ShapeDtypeStruct((n, _D_OUT), jnp.float32),
        scratch_shapes=[
            pltpu.VMEM((_DEPTH, _CHUNK, _D_IN), jnp.float32),
            pltpu.VMEM((_DEPTH, _CHUNK, _D_OUT), jnp.float32),
            pltpu.SemaphoreType.DMA((_DEPTH,)),
            pltpu.SemaphoreType.DMA((_DEPTH,)),
            pltpu.VMEM((_H1A, _D_IN), jnp.float32),
            pltpu.VMEM((1, _H1A), jnp.float32),
            pltpu.VMEM((_H2A, _H1A), jnp.float32),
            pltpu.VMEM((_D_OUT, _H2A), jnp.float32),
        ],
        interpret=interpret,
    )(x, W1, b1.reshape(1, -1), W2, b2.reshape(-1, 1), W3,
      b3.reshape(-1, 1), bn0_mean.reshape(1, -1), bn0_var.reshape(1, -1),
      bn1_mean.reshape(1, -1), bn1_var.reshape(1, -1),
      bn2_mean.reshape(1, -1), bn2_var.reshape(1, -1))


# speedup vs baseline: 1.1013x; 1.1013x over previous
import functools

import jax
import jax.numpy as jnp
from jax.experimental import pallas as pl
from jax.experimental.pallas import tpu as pltpu

_N = 8192
_D_IN = 2048
_D_OUT = 2048
_H1 = 100
_H2 = 50
_EPS = 1e-5
_CHUNK = 512
_DEPTH = 6


def _rm_dot(a, b):
    return jax.lax.dot_general(
        a, b, (((1,), (1,)), ((), ())),
        preferred_element_type=jnp.float32)


def _mlp_pipeline(x_hbm, w1_ref, b1_ref, w2_ref, b2_ref, w3_ref, b3_ref,
                  m0_ref, v0_ref, m1_ref, v1_ref, m2_ref, v2_ref, out_hbm,
                  xbuf, obuf, insems, outsems, w1s, b1s):
    nch = _N // _CHUNK

    def in_copy(c, slot):
        return pltpu.make_async_copy(
            x_hbm.at[pl.ds(c * _CHUNK, _CHUNK), :], xbuf.at[slot],
            insems.at[slot])

    def out_copy(c, slot):
        return pltpu.make_async_copy(
            obuf.at[slot], out_hbm.at[pl.ds(c * _CHUNK, _CHUNK), :],
            outsems.at[slot])

    for s in range(_DEPTH):
        in_copy(s, s).start()

    s0 = jax.lax.rsqrt(v0_ref[...] + _EPS)
    w1s[...] = w1_ref[...] * s0
    b1s[...] = b1_ref[...] - _rm_dot(m0_ref[...] * s0, w1_ref[...])
    s1 = jax.lax.rsqrt(v1_ref[...] + _EPS)
    s2 = jax.lax.rsqrt(v2_ref[...] + _EPS)

    for c in range(nch):
        slot = c % _DEPTH
        in_copy(c, slot).wait()
        if c >= _DEPTH:
            out_copy(c - _DEPTH, slot).wait()
        h = jnp.maximum(_rm_dot(xbuf[slot], w1s[...]) + b1s[...], 0.0)
        h = (h - m1_ref[...]) * s1
        g = jnp.maximum(_rm_dot(h, w2_ref[...]) + b2_ref[...], 0.0)
        g = (g - m2_ref[...]) * s2
        obuf[slot] = _rm_dot(g, w3_ref[...]) + b3_ref[...]
        out_copy(c, slot).start()
        if c + _DEPTH < nch:
            in_copy(c + _DEPTH, slot).start()

    for c in range(nch - _DEPTH, nch):
        out_copy(c, c % _DEPTH).wait()


@functools.partial(jax.jit, static_argnames=("interpret",))
def kernel(x, W1, b1, W2, b2, W3, b3, bn0_mean, bn0_var, bn1_mean, bn1_var,
           bn2_mean, bn2_var, interpret=False):
    n = x.shape[0]
    any_spec = pl.BlockSpec(memory_space=pl.MemorySpace.ANY)
    vmem = pl.BlockSpec(memory_space=pltpu.MemorySpace.VMEM)

    return pl.pallas_call(
        _mlp_pipeline,
        in_specs=[any_spec] + [vmem] * 12,
        out_specs=any_spec,
        out_shape=jax.ShapeDtypeStruct((n, _D_OUT), jnp.float32),
        scratch_shapes=[
            pltpu.VMEM((_DEPTH, _CHUNK, _D_IN), jnp.float32),
            pltpu.VMEM((_DEPTH, _CHUNK, _D_OUT), jnp.float32),
            pltpu.SemaphoreType.DMA((_DEPTH,)),
            pltpu.SemaphoreType.DMA((_DEPTH,)),
            pltpu.VMEM((_H1, _D_IN), jnp.float32),
            pltpu.VMEM((1, _H1), jnp.float32),
        ],
        interpret=interpret,
    )(x, W1, b1.reshape(1, -1), W2, b2.reshape(1, -1), W3,
      b3.reshape(1, -1), bn0_mean.reshape(1, -1), bn0_var.reshape(1, -1),
      bn1_mean.reshape(1, -1), bn1_var.reshape(1, -1),
      bn2_mean.reshape(1, -1), bn2_var.reshape(1, -1))
